# Initial kernel scaffold; baseline (speedup 1.0000x reference)
import jax
import jax.numpy as jnp
from jax.experimental import pallas as pl


def kernel(preds, targets):
    raise NotImplementedError("write your pallas kernel here")


# speedup vs baseline: 41.9767x; 41.9767x over previous
import functools

import jax
import jax.numpy as jnp
import numpy as np
from jax.experimental import pallas as pl
from jax.experimental.pallas import tpu as pltpu

BINS_N = 10
MU_C = 0.02
MMT_C = 0.7
ALPHA_C = 0.25
EPS_C = 1e-5

ROW = 1280
R_BLK = 256
N_ROWS = 64 * 256
N_BLKS = N_ROWS // R_BLK


def _edge_list():
    e = [float(x) / BINS_N for x in range(BINS_N + 1)]
    e[-1] = 1000.0
    return [np.float32(v) for v in e]


def _ghm_block_kernel(p_ref, t_ref, out_ref):
    p = p_ref[...]
    t = t_ref[...]
    shape = p.shape
    col = jax.lax.broadcasted_iota(jnp.int32, shape, 1)
    is_cls = (col % 5) == 0

    u = 2.0 * t - 1.0
    one_m_t = 1.0 - t
    x_t = p * u + one_m_t
    alpha_t = ALPHA_C * u + one_m_t
    om = 1.0 - x_t
    fl = -alpha_t * om * om * jnp.log(x_t + EPS_C)
    focal_part = jnp.sum(jnp.where(is_cls, fl, 0.0))

    v = jnp.where(is_cls & (t > 0.1), 1.0, 0.0)
    tot_part = jnp.sum(v)
    vb = v
    for d in range(1, 5):
        vb = vb + jnp.roll(v, d, axis=1)
    vloc = jnp.where(is_cls, 0.0, vb)

    diff = p - t
    d2 = diff * diff
    root = jnp.sqrt(d2 + MU_C * MU_C)
    loss = root - MU_C
    g = jnp.abs(diff / root)
    vl = vloc * loss

    edges = _edge_list()
    partials = [focal_part, tot_part]
    s_list = [jnp.sum(vloc)]
    l_list = [jnp.sum(vl)]
    for b in range(1, BINS_N):
        m = g >= edges[b]
        s_list.append(jnp.sum(jnp.where(m, vloc, 0.0)))
        l_list.append(jnp.sum(jnp.where(m, vl, 0.0)))
    partials += s_list + l_list

    lane = jax.lax.broadcasted_iota(jnp.int32, (1, 1, 128), 2)
    acc = jnp.zeros((1, 1, 128), jnp.float32)
    for j, val in enumerate(partials):
        acc = acc + jnp.where(lane == j, val, 0.0)
    out_ref[...] = acc


def _streaming_pass(p2d, t2d):
    grid = (N_BLKS,)
    return pl.pallas_call(
        _ghm_block_kernel,
        grid=grid,
        in_specs=[
            pl.BlockSpec((R_BLK, ROW), lambda i: (i, 0)),
            pl.BlockSpec((R_BLK, ROW), lambda i: (i, 0)),
        ],
        out_specs=pl.BlockSpec((1, 1, 128), lambda i: (i, 0, 0)),
        out_shape=jax.ShapeDtypeStruct((N_BLKS, 1, 128), jnp.float32),
        compiler_params=pltpu.CompilerParams(
            dimension_semantics=("arbitrary",),
        ),
    )(p2d, t2d)


@jax.jit
def kernel(preds, targets):
    B, H, W, C = preds.shape
    p2d = preds.reshape(N_ROWS, ROW)
    t2d = targets.reshape(N_ROWS, ROW)
    parts = _streaming_pass(p2d, t2d).sum(axis=(0, 1))

    focal_sum = parts[0]
    tot = jnp.maximum(parts[1], 1.0)
    S = parts[2:2 + BINS_N]
    L = parts[2 + BINS_N:2 + 2 * BINS_N]
    counts = S - jnp.concatenate([S[1:], jnp.zeros((1,), jnp.float32)])
    lsum = L - jnp.concatenate([L[1:], jnp.zeros((1,), jnp.float32)])

    acc_sum = (1.0 - MMT_C) * counts
    n = (counts > 0).astype(jnp.float32).sum()
    per_bin_w = jnp.where(counts > 0, tot / jnp.maximum(acc_sum, 1e-12), 0.0)
    bin_contrib = lsum * per_bin_w
    reg = bin_contrib.sum()
    reg = jnp.where(n > 0, reg / jnp.maximum(n, 1.0), reg)
    reg_loss = reg / tot

    cls_loss = focal_sum / (B * H * W)
    total = cls_loss + reg_loss
    return (total,
            jax.lax.stop_gradient(reg_loss),
            jax.lax.stop_gradient(cls_loss))
